# baseline (speedup 1.0000x reference)
import functools
import jax
import jax.numpy as jnp
from jax import lax
from jax.experimental import pallas as pl
from jax.experimental.pallas import tpu as pltpu

N_NODE = 10000
HID = 32


def _layer1_body(xg_ref, xd_ref, w_ref, out_ref):
    xg = xg_ref[...]
    xd = xd_ref[...]
    w = w_ref[...]
    z1g = jnp.dot(xg, w[0:128, 0:HID], preferred_element_type=jnp.float32)
    d1d = jnp.dot(xd, w[128:256, 0:HID], preferred_element_type=jnp.float32) + w[512, 0:HID]
    z1r = jnp.dot(xd, w[256:384, 0:HID], preferred_element_type=jnp.float32)
    d1g = jnp.dot(xg, w[384:512, 0:HID], preferred_element_type=jnp.float32) + w[512, HID:2 * HID]
    out_ref[...] = jnp.concatenate([z1g, d1d, z1r, d1g], axis=1)


def _layer1_dense(x_gene, x_disease, wpack):
    BLK = 1000
    grid = (N_NODE // BLK,)
    out = pl.pallas_call(
        _layer1_body,
        grid=grid,
        in_specs=[
            pl.BlockSpec((BLK, 128), lambda i: (i, 0)),
            pl.BlockSpec((BLK, 128), lambda i: (i, 0)),
            pl.BlockSpec((520, 4 * HID), lambda i: (0, 0)),
        ],
        out_specs=pl.BlockSpec((BLK, 4 * HID), lambda i: (i, 0)),
        out_shape=jax.ShapeDtypeStruct((N_NODE, 4 * HID), jnp.float32),
    )(x_gene, x_disease, wpack)
    return out[:, 0:HID], out[:, HID:2 * HID], out[:, 2 * HID:3 * HID], out[:, 3 * HID:4 * HID]


def _mid_body(d1d_ref, d1g_ref, sum_d_ref, sum_g_ref, cnt_ref, w2_ref, bn_ref, out_ref):
    cnt = cnt_ref[...]
    inv_d = 1.0 / jnp.maximum(cnt[0:1, :], 1.0)
    inv_g = 1.0 / jnp.maximum(cnt[1:2, :], 1.0)
    h_dis = d1d_ref[...] + sum_d_ref[...] * inv_d.T
    h_gene = d1g_ref[...] + sum_g_ref[...] * inv_g.T

    bnw = bn_ref[...]

    def bn_lrelu(x, g, b):
        m = jnp.mean(x, axis=0, keepdims=True)
        v = jnp.mean((x - m) * (x - m), axis=0, keepdims=True)
        y = (x - m) * lax.rsqrt(v + 1e-5) * g + b
        return jnp.where(y >= 0.0, y, 0.01 * y)

    h_gene = bn_lrelu(h_gene, bnw[0:1, :], bnw[1:2, :])
    h_dis = bn_lrelu(h_dis, bnw[2:3, :], bnw[3:4, :])

    w2 = w2_ref[...]
    z2g = jnp.dot(h_gene, w2[0:HID, 0:HID], preferred_element_type=jnp.float32)
    d2d = jnp.dot(h_dis, w2[HID:2 * HID, 0:HID], preferred_element_type=jnp.float32) + w2[4 * HID, 0:HID]
    z2r = jnp.dot(h_dis, w2[2 * HID:3 * HID, 0:HID], preferred_element_type=jnp.float32)
    d2g = jnp.dot(h_gene, w2[3 * HID:4 * HID, 0:HID], preferred_element_type=jnp.float32) + w2[4 * HID, HID:2 * HID]
    out_ref[...] = jnp.concatenate([z2g, d2d, z2r, d2g], axis=1)


def _mid_dense(d1d, d1g, sum_d, sum_g, cnt2, w2pack, bnpack):
    out = pl.pallas_call(
        _mid_body,
        in_specs=[pl.BlockSpec(x.shape, lambda: tuple(0 for _ in x.shape))
                  for x in (d1d, d1g, sum_d, sum_g, cnt2, w2pack, bnpack)],
        out_specs=pl.BlockSpec((N_NODE, 4 * HID), lambda: (0, 0)),
        out_shape=jax.ShapeDtypeStruct((N_NODE, 4 * HID), jnp.float32),
    )(d1d, d1g, sum_d, sum_g, cnt2, w2pack, bnpack)
    return out[:, 0:HID], out[:, HID:2 * HID], out[:, 2 * HID:3 * HID], out[:, 3 * HID:4 * HID]


def _final_body(d2d_ref, d2g_ref, sum_d_ref, sum_g_ref, cnt_ref, out_ref):
    cnt = cnt_ref[...]
    inv_d = 1.0 / jnp.maximum(cnt[0:1, :], 1.0)
    inv_g = 1.0 / jnp.maximum(cnt[1:2, :], 1.0)
    h2_dis = d2d_ref[...] + sum_d_ref[...] * inv_d.T
    h2_gene = d2g_ref[...] + sum_g_ref[...] * inv_g.T
    out_ref[...] = jnp.concatenate([h2_gene, h2_dis], axis=1)


def _final_dense(d2d, d2g, sum_d, sum_g, cnt2):
    out = pl.pallas_call(
        _final_body,
        in_specs=[pl.BlockSpec(x.shape, lambda: tuple(0 for _ in x.shape))
                  for x in (d2d, d2g, sum_d, sum_g, cnt2)],
        out_specs=pl.BlockSpec((N_NODE, 2 * HID), lambda: (0, 0)),
        out_shape=jax.ShapeDtypeStruct((N_NODE, 2 * HID), jnp.float32),
    )(d2d, d2g, sum_d, sum_g, cnt2)
    return out[:, 0:HID], out[:, HID:2 * HID]


def _seg_sum(feat, es, ed, n):
    g = jnp.take(feat, es, axis=0)
    return jax.ops.segment_sum(g, ed, num_segments=n)


def kernel(x_gene, x_disease, edge_index_gda, edge_index_rev, edge_label_index_gda, edge_label_index_rev,
           W1_gda_src, b1_gda_src, W1_gda_dst, b1_gda_dst, W1_gda_upd, b1_gda_upd,
           W1_rev_src, b1_rev_src, W1_rev_dst, b1_rev_dst, W1_rev_upd, b1_rev_upd,
           W2_gda_src, b2_gda_src, W2_gda_dst, b2_gda_dst, W2_gda_upd, b2_gda_upd,
           W2_rev_src, b2_rev_src, W2_rev_dst, b2_rev_dst, W2_rev_upd, b2_rev_upd,
           bn_gene_g, bn_gene_b, bn_dis_g, bn_dis_b):
    def fold(Ws, bs, Wd, bd, Wu, bu):
        Wud, Wus = Wu[:, :HID], Wu[:, HID:]
        return Wud @ Wd, Wus @ Ws, Wud @ bd + Wus @ bs + bu

    A1g_d, A1g_s, c1g = fold(W1_gda_src, b1_gda_src, W1_gda_dst, b1_gda_dst, W1_gda_upd, b1_gda_upd)
    A1r_d, A1r_s, c1r = fold(W1_rev_src, b1_rev_src, W1_rev_dst, b1_rev_dst, W1_rev_upd, b1_rev_upd)
    A2g_d, A2g_s, c2g = fold(W2_gda_src, b2_gda_src, W2_gda_dst, b2_gda_dst, W2_gda_upd, b2_gda_upd)
    A2r_d, A2r_s, c2r = fold(W2_rev_src, b2_rev_src, W2_rev_dst, b2_rev_dst, W2_rev_upd, b2_rev_upd)

    wpack = jnp.zeros((520, 4 * HID), jnp.float32)
    wpack = wpack.at[0:128, 0:HID].set(A1g_s.T)
    wpack = wpack.at[128:256, 0:HID].set(A1g_d.T)
    wpack = wpack.at[256:384, 0:HID].set(A1r_s.T)
    wpack = wpack.at[384:512, 0:HID].set(A1r_d.T)
    wpack = wpack.at[512, 0:HID].set(c1g)
    wpack = wpack.at[512, HID:2 * HID].set(c1r)

    w2pack = jnp.zeros((4 * HID + 8, 4 * HID), jnp.float32)
    w2pack = w2pack.at[0:HID, 0:HID].set(A2g_s.T)
    w2pack = w2pack.at[HID:2 * HID, 0:HID].set(A2g_d.T)
    w2pack = w2pack.at[2 * HID:3 * HID, 0:HID].set(A2r_s.T)
    w2pack = w2pack.at[3 * HID:4 * HID, 0:HID].set(A2r_d.T)
    w2pack = w2pack.at[4 * HID, 0:HID].set(c2g)
    w2pack = w2pack.at[4 * HID, HID:2 * HID].set(c2r)

    bnpack = jnp.stack([bn_gene_g, bn_gene_b, bn_dis_g, bn_dis_b], axis=0)

    eg_s, eg_d = edge_index_gda[0].astype(jnp.int32), edge_index_gda[1].astype(jnp.int32)
    er_s, er_d = edge_index_rev[0].astype(jnp.int32), edge_index_rev[1].astype(jnp.int32)

    z1g, d1d, z1r, d1g = _layer1_dense(x_gene, x_disease, wpack)

    ones = jnp.ones(eg_d.shape, jnp.float32)
    cnt_d = jax.ops.segment_sum(ones, eg_d, num_segments=N_NODE)
    cnt_g = jax.ops.segment_sum(ones, er_d, num_segments=N_NODE)
    cnt2 = jnp.stack([cnt_d, cnt_g], axis=0)

    sum1_d = _seg_sum(z1g, eg_s, eg_d, N_NODE)
    sum1_g = _seg_sum(z1r, er_s, er_d, N_NODE)

    z2g, d2d, z2r, d2g = _mid_dense(d1d, d1g, sum1_d, sum1_g, cnt2, w2pack, bnpack)

    sum2_d = _seg_sum(z2g, eg_s, eg_d, N_NODE)
    sum2_g = _seg_sum(z2r, er_s, er_d, N_NODE)

    h2_gene, h2_dis = _final_dense(d2d, d2g, sum2_d, sum2_g, cnt2)

    eli = edge_label_index_gda
    pred_gda = jnp.sum(jnp.take(h2_gene, eli[0], axis=0) * jnp.take(h2_dis, eli[1], axis=0), axis=-1)
    eli2 = edge_label_index_rev
    pred_rev = jnp.sum(jnp.take(h2_dis, eli2[0], axis=0) * jnp.take(h2_gene, eli2[1], axis=0), axis=-1)
    return (pred_gda, pred_rev)


# speedup vs baseline: 9.6969x; 9.6969x over previous
import functools
import jax
import jax.numpy as jnp
from jax import lax
from jax.experimental import pallas as pl
from jax.experimental.pallas import tpu as pltpu
from jax.experimental.pallas import tpu_sc as plsc

N_NODE = 10000
HID = 32
N_EDGE = 320000
N_LBL = 100000

NC = 2
NS = 16
NW = NC * NS
CH = 128
ROWS_PT = 640
N_PAD = ROWS_PT * NS

ECH = (N_EDGE + NW * CH - 1) // (NW * CH)
ECH = ECH + (ECH % 2)
E_PAD = NW * ECH * CH
TOT_CH = NW * ECH
C0, C1 = 80, 80
assert NS * (C0 + C1) == TOT_CH and C0 % 2 == 0 and C1 % 2 == 0
LCH = (N_LBL + NW * CH - 1) // (NW * CH)
LCH = LCH + (LCH % 2)
L_PAD = NW * LCH * CH
L_PW = LCH * CH

_mesh = plsc.VectorSubcoreMesh(core_axis_name="c", subcore_axis_name="s")


def _wid():
    return lax.axis_index("s") * NC + lax.axis_index("c")


def _zero_rows(buf, nrows):
    z = jnp.zeros((16,), jnp.float32)

    def body(i, carry):
        buf[i, pl.ds(0, 16)] = z
        buf[i, pl.ds(16, 16)] = z
        return carry

    lax.fori_loop(0, nrows, body, 0)


@functools.partial(
    pl.kernel,
    out_type=jax.ShapeDtypeStruct((2, NW, N_PAD), jnp.float32),
    mesh=_mesh,
    scratch_types=[
        pltpu.VMEM((N_PAD,), jnp.float32),
        pltpu.VMEM((ECH, CH), jnp.int32),
    ],
    compiler_params=pltpu.CompilerParams(needs_layout_passes=False),
)
def _count_kernel(dst_gda, dst_rev, out, hist, didx):
    wid = _wid()
    ones = jnp.ones((16,), jnp.float32)
    z = jnp.zeros((16,), jnp.float32)

    def zbody(i, c):
        hist[pl.ds(i * 16, 16)] = z
        return c

    for s, dref in ((0, dst_gda), (1, dst_rev)):
        lax.fori_loop(0, N_PAD // 16, zbody, 0)
        pltpu.sync_copy(dref.at[pl.ds(wid * ECH, ECH)], didx)

        def cbody(j, c):
            for g in range(CH // 16):
                idx = didx[j, pl.ds(g * 16, 16)]
                plsc.addupdate_scatter(hist, [idx], ones)
            return c

        lax.fori_loop(0, ECH, cbody, 0)
        pltpu.sync_copy(hist, out.at[s, wid])


@functools.partial(
    pl.kernel,
    out_type=jax.ShapeDtypeStruct((NC, 2, N_PAD, HID), jnp.float32),
    mesh=_mesh,
    scratch_types=[
        pltpu.VMEM_SHARED((2, N_PAD, HID), jnp.float32),
        pltpu.VMEM((ROWS_PT, HID), jnp.float32),
        pltpu.VMEM((C0, CH), jnp.int32),
        pltpu.VMEM((C0, CH), jnp.int32),
        pltpu.VMEM((CH, HID), jnp.float32),
        pltpu.VMEM((CH, HID), jnp.float32),
        pltpu.SemaphoreType.DMA,
        pltpu.SemaphoreType.DMA,
        pltpu.SemaphoreType.DMA,
        pltpu.SemaphoreType.DMA,
    ],
    compiler_params=pltpu.CompilerParams(use_tc_tiling_on_sc=False, needs_layout_passes=False),
)
def _agg_kernel(tab_gda, tab_rev, src_gda, dst_gda, src_rev, dst_rev, histdep,
                out, acc, zbuf, sidx, didx, rows0, rows1, gs0, gs1, ss0, ss1):
    cid = lax.axis_index("c")
    sid = lax.axis_index("s")

    _zero_rows(zbuf, ROWS_PT)
    for s in range(2):
        pltpu.sync_copy(zbuf, acc.at[s, pl.ds(sid * ROWS_PT, ROWS_PT)])
    plsc.subcore_barrier()

    for s, tab, sref, dref in ((0, tab_gda, src_gda, dst_gda),
                               (1, tab_rev, src_rev, dst_rev)):
        acc_s = acc.at[s]

        def run(C, start):
            pltpu.sync_copy(sref.at[pl.ds(start, C)], sidx.at[pl.ds(0, C)])
            pltpu.sync_copy(dref.at[pl.ds(start, C)], didx.at[pl.ds(0, C)])

            pltpu.async_copy(tab.at[sidx.at[0]], rows0, gs0)

            def body(jj, c):
                j0 = jj * 2
                pltpu.make_async_copy(tab.at[sidx.at[j0]], rows0, gs0).wait()

                @pl.when(jj > 0)
                def _():
                    pltpu.make_async_copy(rows1, acc_s.at[didx.at[j0 - 1]], ss1).wait()

                pltpu.async_copy(tab.at[sidx.at[j0 + 1]], rows1, gs1)
                pltpu.async_copy(rows0, acc_s.at[didx.at[j0]], ss0, add=True)
                pltpu.make_async_copy(tab.at[sidx.at[j0 + 1]], rows1, gs1).wait()
                pltpu.async_copy(rows1, acc_s.at[didx.at[j0 + 1]], ss1, add=True)
                pltpu.make_async_copy(rows0, acc_s.at[didx.at[j0]], ss0).wait()

                @pl.when(jj + 1 < C // 2)
                def _():
                    pltpu.async_copy(tab.at[sidx.at[j0 + 2]], rows0, gs0)

                return c

            lax.fori_loop(0, C // 2, body, 0)
            pltpu.make_async_copy(rows1, acc_s.at[didx.at[C - 1]], ss1).wait()

        run(C0, (cid * NS + sid) * C0)

    plsc.subcore_barrier()

    for s in range(2):
        pltpu.sync_copy(acc.at[s, pl.ds(sid * ROWS_PT, ROWS_PT)],
                        out.at[cid, s, pl.ds(sid * ROWS_PT, ROWS_PT)])


@functools.partial(
    pl.kernel,
    out_type=[jax.ShapeDtypeStruct((L_PAD,), jnp.float32),
              jax.ShapeDtypeStruct((L_PAD,), jnp.float32)],
    mesh=_mesh,
    scratch_types=[
        pltpu.VMEM((LCH, CH), jnp.int32),
        pltpu.VMEM((LCH, CH), jnp.int32),
        pltpu.VMEM((CH, HID), jnp.float32),
        pltpu.VMEM((CH, HID), jnp.float32),
        pltpu.VMEM((CH, HID), jnp.float32),
        pltpu.VMEM((CH, HID), jnp.float32),
        pltpu.VMEM((L_PW,), jnp.float32),
        pltpu.SemaphoreType.DMA,
        pltpu.SemaphoreType.DMA,
        pltpu.SemaphoreType.DMA,
        pltpu.SemaphoreType.DMA,
    ],
    compiler_params=pltpu.CompilerParams(use_tc_tiling_on_sc=False, needs_layout_passes=False),
)
def _pred_kernel(h2_gene, h2_dis, ia_gda, ib_gda, ia_rev, ib_rev,
                 out_gda, out_rev, iav, ibv, ra0, rb0, ra1, rb1, dots,
                 sa0, sb0, sa1, sb1):
    wid = _wid()
    lane = lax.iota(jnp.int32, 16)

    def chunk_dots(j, ra, rb):
        def gbody(g, c2):
            acc16 = jnp.zeros((16,), jnp.float32)
            rows16 = lane + g * 16
            for d in range(HID):
                col = (lane + d) & (HID - 1)
                va = plsc.load_gather(ra, [rows16, col])
                vb = plsc.load_gather(rb, [rows16, col])
                acc16 = acc16 + va * vb
            dots[pl.ds(j * CH + g * 16, 16)] = acc16
            return c2

        lax.fori_loop(0, CH // 16, gbody, 0)

    for ta, tb, ia, ib, out in ((h2_gene, h2_dis, ia_gda, ib_gda, out_gda),
                                (h2_dis, h2_gene, ia_rev, ib_rev, out_rev)):
        pltpu.sync_copy(ia.at[wid], iav)
        pltpu.sync_copy(ib.at[wid], ibv)

        pltpu.async_copy(ta.at[iav.at[0]], ra0, sa0)
        pltpu.async_copy(tb.at[ibv.at[0]], rb0, sb0)

        def body(jj, c):
            j0 = jj * 2
            pltpu.make_async_copy(ta.at[iav.at[j0]], ra0, sa0).wait()
            pltpu.make_async_copy(tb.at[ibv.at[j0]], rb0, sb0).wait()
            pltpu.async_copy(ta.at[iav.at[j0 + 1]], ra1, sa1)
            pltpu.async_copy(tb.at[ibv.at[j0 + 1]], rb1, sb1)
            chunk_dots(j0, ra0, rb0)
            pltpu.make_async_copy(ta.at[iav.at[j0 + 1]], ra1, sa1).wait()
            pltpu.make_async_copy(tb.at[ibv.at[j0 + 1]], rb1, sb1).wait()

            @pl.when(jj + 1 < LCH // 2)
            def _():
                pltpu.async_copy(ta.at[iav.at[j0 + 2]], ra0, sa0)
                pltpu.async_copy(tb.at[ibv.at[j0 + 2]], rb0, sb0)

            chunk_dots(j0 + 1, ra1, rb1)
            return c

        lax.fori_loop(0, LCH // 2, body, 0)
        pltpu.sync_copy(dots, out.at[pl.ds(wid * L_PW, L_PW)])


def _layer1_body(xg_ref, xd_ref, w_ref, out_ref):
    xg = xg_ref[...]
    xd = xd_ref[...]
    w = w_ref[...]
    z1g = jnp.dot(xg, w[0:128, 0:HID], preferred_element_type=jnp.float32)
    d1d = jnp.dot(xd, w[128:256, 0:HID], preferred_element_type=jnp.float32) + w[512, 0:HID]
    z1r = jnp.dot(xd, w[256:384, 0:HID], preferred_element_type=jnp.float32)
    d1g = jnp.dot(xg, w[384:512, 0:HID], preferred_element_type=jnp.float32) + w[512, HID:2 * HID]
    out_ref[...] = jnp.concatenate([z1g, d1d, z1r, d1g], axis=1)


def _layer1_dense(x_gene, x_disease, wpack):
    BLK = 1000
    out = pl.pallas_call(
        _layer1_body,
        grid=(N_NODE // BLK,),
        in_specs=[
            pl.BlockSpec((BLK, 128), lambda i: (i, 0)),
            pl.BlockSpec((BLK, 128), lambda i: (i, 0)),
            pl.BlockSpec((520, 4 * HID), lambda i: (0, 0)),
        ],
        out_specs=pl.BlockSpec((BLK, 4 * HID), lambda i: (i, 0)),
        out_shape=jax.ShapeDtypeStruct((N_NODE, 4 * HID), jnp.float32),
    )(x_gene, x_disease, wpack)
    return out[:, 0:HID], out[:, HID:2 * HID], out[:, 2 * HID:3 * HID], out[:, 3 * HID:4 * HID]


def _mid_body(d1d_ref, d1g_ref, psum_ref, cnt_ref, w2_ref, bn_ref, out_ref):
    cnt = jnp.sum(cnt_ref[...], axis=1)[:, :N_NODE]
    inv_d = 1.0 / jnp.maximum(cnt[0:1, :], 1.0)
    inv_g = 1.0 / jnp.maximum(cnt[1:2, :], 1.0)
    psum = psum_ref[...]
    h_dis = d1d_ref[...] + (psum[0, 0, :N_NODE] + psum[1, 0, :N_NODE]) * inv_d.T
    h_gene = d1g_ref[...] + (psum[0, 1, :N_NODE] + psum[1, 1, :N_NODE]) * inv_g.T

    bnw = bn_ref[...]

    def bn_lrelu(x, g, b):
        m = jnp.mean(x, axis=0, keepdims=True)
        v = jnp.mean((x - m) * (x - m), axis=0, keepdims=True)
        y = (x - m) / jnp.sqrt(v + 1e-5) * g + b
        return jnp.where(y >= 0.0, y, 0.01 * y)

    h_gene = bn_lrelu(h_gene, bnw[0:1, :], bnw[1:2, :])
    h_dis = bn_lrelu(h_dis, bnw[2:3, :], bnw[3:4, :])

    w2 = w2_ref[...]
    z2g = jnp.dot(h_gene, w2[0:HID, 0:HID], preferred_element_type=jnp.float32)
    d2d = jnp.dot(h_dis, w2[HID:2 * HID, 0:HID], preferred_element_type=jnp.float32) + w2[4 * HID, 0:HID]
    z2r = jnp.dot(h_dis, w2[2 * HID:3 * HID, 0:HID], preferred_element_type=jnp.float32)
    d2g = jnp.dot(h_gene, w2[3 * HID:4 * HID, 0:HID], preferred_element_type=jnp.float32) + w2[4 * HID, HID:2 * HID]
    out_ref[...] = jnp.concatenate([z2g, d2d, z2r, d2g], axis=1)


def _mid_dense(d1d, d1g, psum, cnt2, w2pack, bnpack):
    out = pl.pallas_call(
        _mid_body,
        in_specs=[pl.BlockSpec(x.shape, functools.partial(lambda r: (0,) * r, len(x.shape)))
                  for x in (d1d, d1g, psum, cnt2, w2pack, bnpack)],
        out_specs=pl.BlockSpec((N_NODE, 4 * HID), lambda: (0, 0)),
        out_shape=jax.ShapeDtypeStruct((N_NODE, 4 * HID), jnp.float32),
    )(d1d, d1g, psum, cnt2, w2pack, bnpack)
    return out[:, 0:HID], out[:, HID:2 * HID], out[:, 2 * HID:3 * HID], out[:, 3 * HID:4 * HID]


def _final_body(d2d_ref, d2g_ref, psum_ref, cnt_ref, out_ref):
    cnt = jnp.sum(cnt_ref[...], axis=1)[:, :N_NODE]
    inv_d = 1.0 / jnp.maximum(cnt[0:1, :], 1.0)
    inv_g = 1.0 / jnp.maximum(cnt[1:2, :], 1.0)
    psum = psum_ref[...]
    h2_dis = d2d_ref[...] + (psum[0, 0, :N_NODE] + psum[1, 0, :N_NODE]) * inv_d.T
    h2_gene = d2g_ref[...] + (psum[0, 1, :N_NODE] + psum[1, 1, :N_NODE]) * inv_g.T
    out_ref[...] = jnp.concatenate([h2_gene, h2_dis], axis=1)


def _final_dense(d2d, d2g, psum, cnt2):
    out = pl.pallas_call(
        _final_body,
        in_specs=[pl.BlockSpec(x.shape, functools.partial(lambda r: (0,) * r, len(x.shape)))
                  for x in (d2d, d2g, psum, cnt2)],
        out_specs=pl.BlockSpec((N_NODE, 2 * HID), lambda: (0, 0)),
        out_shape=jax.ShapeDtypeStruct((N_NODE, 2 * HID), jnp.float32),
    )(d2d, d2g, psum, cnt2)
    return out[:, 0:HID], out[:, HID:2 * HID]


def _interleave(x):
    return x.reshape(ECH, NW, CH).transpose(1, 0, 2).reshape(TOT_CH, CH)


def _prep_edges(ei, n_pad):
    src = ei[0].astype(jnp.int32)
    dst = ei[1].astype(jnp.int32)
    pad = n_pad - src.shape[0]
    ar = jnp.arange(pad, dtype=jnp.int32)
    src_agg = jnp.concatenate([src, N_NODE + (ar % 16)]).reshape(-1, CH)
    dst_agg = jnp.concatenate([dst, (ar * 131) % N_PAD]).reshape(-1, CH)
    dst_cnt = jnp.concatenate([dst, N_NODE + (ar % (N_PAD - N_NODE))]).reshape(-1, CH)
    return _interleave(src_agg), _interleave(dst_agg), _interleave(dst_cnt)


def kernel(x_gene, x_disease, edge_index_gda, edge_index_rev, edge_label_index_gda, edge_label_index_rev,
           W1_gda_src, b1_gda_src, W1_gda_dst, b1_gda_dst, W1_gda_upd, b1_gda_upd,
           W1_rev_src, b1_rev_src, W1_rev_dst, b1_rev_dst, W1_rev_upd, b1_rev_upd,
           W2_gda_src, b2_gda_src, W2_gda_dst, b2_gda_dst, W2_gda_upd, b2_gda_upd,
           W2_rev_src, b2_rev_src, W2_rev_dst, b2_rev_dst, W2_rev_upd, b2_rev_upd,
           bn_gene_g, bn_gene_b, bn_dis_g, bn_dis_b):
    def fold(Ws, bs, Wd, bd, Wu, bu):
        Wud, Wus = Wu[:, :HID], Wu[:, HID:]
        return Wud @ Wd, Wus @ Ws, Wud @ bd + Wus @ bs + bu

    A1g_d, A1g_s, c1g = fold(W1_gda_src, b1_gda_src, W1_gda_dst, b1_gda_dst, W1_gda_upd, b1_gda_upd)
    A1r_d, A1r_s, c1r = fold(W1_rev_src, b1_rev_src, W1_rev_dst, b1_rev_dst, W1_rev_upd, b1_rev_upd)
    A2g_d, A2g_s, c2g = fold(W2_gda_src, b2_gda_src, W2_gda_dst, b2_gda_dst, W2_gda_upd, b2_gda_upd)
    A2r_d, A2r_s, c2r = fold(W2_rev_src, b2_rev_src, W2_rev_dst, b2_rev_dst, W2_rev_upd, b2_rev_upd)

    wpack = jnp.zeros((520, 4 * HID), jnp.float32)
    wpack = wpack.at[0:128, 0:HID].set(A1g_s.T)
    wpack = wpack.at[128:256, 0:HID].set(A1g_d.T)
    wpack = wpack.at[256:384, 0:HID].set(A1r_s.T)
    wpack = wpack.at[384:512, 0:HID].set(A1r_d.T)
    wpack = wpack.at[512, 0:HID].set(c1g)
    wpack = wpack.at[512, HID:2 * HID].set(c1r)

    w2pack = jnp.zeros((4 * HID + 8, 4 * HID), jnp.float32)
    w2pack = w2pack.at[0:HID, 0:HID].set(A2g_s.T)
    w2pack = w2pack.at[HID:2 * HID, 0:HID].set(A2g_d.T)
    w2pack = w2pack.at[2 * HID:3 * HID, 0:HID].set(A2r_s.T)
    w2pack = w2pack.at[3 * HID:4 * HID, 0:HID].set(A2r_d.T)
    w2pack = w2pack.at[4 * HID, 0:HID].set(c2g)
    w2pack = w2pack.at[4 * HID, HID:2 * HID].set(c2r)

    bnpack = jnp.stack([bn_gene_g, bn_gene_b, bn_dis_g, bn_dis_b], axis=0)

    sg, dg, cg = _prep_edges(edge_index_gda, E_PAD)
    sr, dr, cr = _prep_edges(edge_index_rev, E_PAD)

    hist = _count_kernel(cg, cr)

    z1g, d1d, z1r, d1g = _layer1_dense(x_gene, x_disease, wpack)

    zpad = ((0, 16), (0, 0))
    psum1 = _agg_kernel(jnp.pad(z1g, zpad), jnp.pad(z1r, zpad), sg, dg, sr, dr, hist)
    z2g, d2d, z2r, d2g = _mid_dense(d1d, d1g, psum1, hist, w2pack, bnpack)

    psum2 = _agg_kernel(jnp.pad(z2g, zpad), jnp.pad(z2r, zpad), sg, dg, sr, dr, hist)
    h2_gene, h2_dis = _final_dense(d2d, d2g, psum2, hist)

    def prep_lbl(eli):
        ia = jnp.pad(eli[0].astype(jnp.int32), (0, L_PAD - N_LBL)).reshape(NW, LCH, CH)
        ib = jnp.pad(eli[1].astype(jnp.int32), (0, L_PAD - N_LBL)).reshape(NW, LCH, CH)
        return ia, ib

    iag, ibg = prep_lbl(edge_label_index_gda)
    iar, ibr = prep_lbl(edge_label_index_rev)
    pg, pr = _pred_kernel(h2_gene, h2_dis, iag, ibg, iar, ibr)
    return (pg[:N_LBL], pr[:N_LBL])
